# Initial kernel scaffold; baseline (speedup 1.0000x reference)
import jax
import jax.numpy as jnp
from jax.experimental import pallas as pl


def kernel(data):
    raise NotImplementedError("write your pallas kernel here")


# speedup vs baseline: 2.9783x; 2.9783x over previous
import jax
import jax.numpy as jnp
from jax import lax
from jax.experimental import pallas as pl
from jax.experimental.pallas import tpu as pltpu

_N = 8192
_D = 256
_K = 64
_ITERS = 20
def _dot(a, b, dims, prec=lax.Precision.HIGHEST):
    return lax.dot_general(a, b, (dims, ((), ())),
                           preferred_element_type=jnp.float32,
                           precision=prec)


def _kmeans_kernel(x_ref, g_ref, c0_ref, cent_ref, lab_ref):
    x = x_ref[:]
    gmat = g_ref[:]
    c0 = c0_ref[:]

    ones_row = jnp.ones((1, _D), jnp.float32)
    x2r = _dot(ones_row, x * x, ((1,), (1,)))

    lane_n = lax.broadcasted_iota(jnp.int32, (1, _N), 1)
    lane_g = lax.broadcasted_iota(jnp.int32, (1, _K - 1), 1)
    row_k = lax.broadcasted_iota(jnp.int32, (_K, 1), 0)

    def dist_row(c_row):
        xc = _dot(c_row, x, ((1,), (1,)), lax.Precision.DEFAULT)
        c2 = jnp.sum(c_row * c_row)
        return jnp.maximum(x2r - 2.0 * xc + c2, 0.0)

    def gather_row(idx):
        oh = jnp.where(lane_n == idx, 1.0, 0.0)
        return _dot(oh, x, ((1,), (0,)))

    c_init = jnp.where(row_k == 0, c0, 0.0)

    def init_step(i, carry):
        c, md = carry
        ohg = jnp.where(lane_g == i - 1, 1.0, 0.0)
        g = _dot(ohg, gmat, ((1,), (0,)))
        dist_sum = jnp.sum(md) + 1e-10
        score = g + jnp.log(md / dist_sum)
        m = jnp.max(score)
        idx = jnp.min(jnp.where(score == m, lane_n, _N))
        row = gather_row(idx)
        c = jnp.where(row_k == i, row, c)
        md = jnp.minimum(md, dist_row(row))
        return c, md

    c, _ = lax.fori_loop(1, _K, init_step, (c_init, dist_row(c0)))

    def assign(c):
        cx = _dot(c, x, ((1,), (1,)), lax.Precision.DEFAULT)
        c2 = jnp.sum(c * c, axis=1, keepdims=True)
        d = jnp.maximum(x2r - 2.0 * cx + c2, 0.0)
        md = jnp.min(d, axis=0, keepdims=True)
        lab = jnp.min(jnp.where(d == md, row_k, _K), axis=0, keepdims=True)
        return md, lab

    def lloyd_step(_, c):
        md, lab = assign(c)
        fm = jnp.max(md)
        fidx = jnp.min(jnp.where(md == fm, lane_n, _N))
        frow = gather_row(fidx)
        oh = jnp.where(lab == row_k, 1.0, 0.0)
        sums = _dot(oh, x, ((1,), (0,)))
        counts = jnp.sum(oh, axis=1, keepdims=True)
        safe = sums / jnp.maximum(counts, 1.0)
        return jnp.where(counts > 0, safe, frow)

    c = lax.fori_loop(0, _ITERS, lloyd_step, c)
    _, lab = assign(c)
    cent_ref[:] = c
    lab_ref[:] = lab


def _pallas_kmeans(flat, gmat, c0):
    return pl.pallas_call(
        _kmeans_kernel,
        out_shape=(jax.ShapeDtypeStruct((_K, _D), jnp.float32),
                   jax.ShapeDtypeStruct((1, _N), jnp.int32)),
        compiler_params=pltpu.CompilerParams(
            vmem_limit_bytes=100 * 1024 * 1024),
    )(flat, gmat, c0)


def kernel(data):
    flat = data.reshape(_N, _D)
    init_key = jax.random.PRNGKey(0)
    first = jax.random.choice(init_key, jnp.arange(_N), (1,), replace=False)
    c0 = flat[first[0]][None, :]
    gmat = jnp.stack([
        jax.random.gumbel(jax.random.fold_in(init_key, i), (_N,), jnp.float32)
        for i in range(1, _K)])
    cent, lab = _pallas_kmeans(flat, gmat, c0)
    return cent, lab.reshape(_N)


# speedup vs baseline: 4.2416x; 1.4242x over previous
import jax
import jax.numpy as jnp
from jax import lax
from jax.experimental import pallas as pl
from jax.experimental.pallas import tpu as pltpu

_N = 8192
_D = 256
_K = 64
_ITERS = 20
def _dot(a, b, dims, prec=lax.Precision.HIGHEST):
    return lax.dot_general(a, b, (dims, ((), ())),
                           preferred_element_type=jnp.float32,
                           precision=prec)


def _kmeans_kernel(x_ref, g_ref, c0_ref, cent_ref, lab_ref):
    x = x_ref[:]
    gmat = g_ref[:]
    c0 = c0_ref[:]

    ones_row = jnp.ones((1, _D), jnp.float32)
    x2r = _dot(ones_row, x * x, ((1,), (1,)))

    lane_n = lax.broadcasted_iota(jnp.int32, (1, _N), 1)
    lane_g = lax.broadcasted_iota(jnp.int32, (1, _K - 1), 1)
    row_k = lax.broadcasted_iota(jnp.int32, (_K, 1), 0)

    def dist_row(c_row):
        xc = _dot(c_row, x, ((1,), (1,)), lax.Precision.DEFAULT)
        c2 = jnp.sum(c_row * c_row)
        return jnp.maximum(x2r - 2.0 * xc + c2, 0.0)

    def gather_row(idx):
        oh = jnp.where(lane_n == idx, 1.0, 0.0)
        return _dot(oh, x, ((1,), (0,)))

    c_init = jnp.where(row_k == 0, c0, 0.0)

    def init_step(i, carry):
        c, md = carry
        ohg = jnp.where(lane_g == i - 1, 1.0, 0.0)
        g = _dot(ohg, gmat, ((1,), (0,)))
        dist_sum = jnp.sum(md) + 1e-10
        score = g + jnp.log(md / dist_sum)
        m = jnp.max(score)
        idx = jnp.min(jnp.where(score == m, lane_n, _N))
        row = gather_row(idx)
        c = jnp.where(row_k == i, row, c)
        md = jnp.minimum(md, dist_row(row))
        return c, md

    c, _ = lax.fori_loop(1, _K, init_step, (c_init, dist_row(c0)))

    def assign(c):
        cx = _dot(c, x, ((1,), (1,)), lax.Precision.DEFAULT)
        c2 = jnp.sum(c * c, axis=1, keepdims=True)
        d = jnp.maximum(x2r - 2.0 * cx + c2, 0.0)
        md = jnp.min(d, axis=0, keepdims=True)
        lab = jnp.min(jnp.where(d == md, row_k, _K), axis=0, keepdims=True)
        return md, lab

    def lloyd_step(_, c):
        md, lab = assign(c)
        fm = jnp.max(md)
        fidx = jnp.min(jnp.where(md == fm, lane_n, _N))
        frow = gather_row(fidx)
        oh = jnp.where(lab == row_k, 1.0, 0.0)
        sums = _dot(oh, x, ((1,), (0,)))
        counts = jnp.sum(oh, axis=1, keepdims=True)
        safe = sums / jnp.maximum(counts, 1.0)
        return jnp.where(counts > 0, safe, frow)

    c = lax.fori_loop(0, _ITERS, lloyd_step, c)
    _, lab = assign(c)
    cent_ref[:] = c
    lab_ref[:] = lab


def _pallas_kmeans(flat, gmat, c0):
    return pl.pallas_call(
        _kmeans_kernel,
        out_shape=(jax.ShapeDtypeStruct((_K, _D), jnp.float32),
                   jax.ShapeDtypeStruct((1, _N), jnp.int32)),
        compiler_params=pltpu.CompilerParams(
            vmem_limit_bytes=100 * 1024 * 1024),
    )(flat, gmat, c0)


def kernel(data):
    flat = data.reshape(_N, _D)
    with jax.ensure_compile_time_eval():
        init_key = jax.random.PRNGKey(0)
        first = int(jax.random.choice(init_key, jnp.arange(_N), (1,),
                                      replace=False)[0])
        gmat = jnp.stack([
            jax.random.gumbel(jax.random.fold_in(init_key, i), (_N,),
                              jnp.float32)
            for i in range(1, _K)])
    c0 = flat[first][None, :]
    cent, lab = _pallas_kmeans(flat, gmat, c0)
    return cent, lab.reshape(_N)


# speedup vs baseline: 11.7481x; 2.7697x over previous
import jax
import jax.numpy as jnp
from jax import lax
from jax.experimental import pallas as pl
from jax.experimental.pallas import tpu as pltpu

_N = 8192
_D = 256
_K = 64
_ITERS = 20
def _dot(a, b, dims, prec=lax.Precision.HIGHEST):
    return lax.dot_general(a, b, (dims, ((), ())),
                           preferred_element_type=jnp.float32,
                           precision=prec)


def _kmeans_kernel(x_ref, g_ref, c0_ref, cent_ref, lab_ref):
    x = x_ref[:]
    c0 = c0_ref[:]

    ones_row = jnp.ones((1, _D), jnp.float32)
    x2r = _dot(ones_row, x * x, ((1,), (1,)))

    lane_n = lax.broadcasted_iota(jnp.int32, (1, _N), 1)
    row_k = lax.broadcasted_iota(jnp.int32, (_K, 1), 0)

    def dist_row(c_row):
        xc = _dot(c_row, x, ((1,), (1,)), lax.Precision.DEFAULT)
        c2 = jnp.sum(c_row * c_row)
        return jnp.maximum(x2r - 2.0 * xc + c2, 0.0)

    def gather_row(idx):
        return x_ref[pl.ds(idx, 1), :]

    c_init = jnp.where(row_k == 0, c0, 0.0)

    def init_step(i, carry):
        c, md = carry
        g = g_ref[pl.ds(i - 1, 1), :]
        dist_sum = jnp.sum(md) + 1e-10
        score = g + jnp.log(md / dist_sum)
        m = jnp.max(score)
        idx = jnp.min(jnp.where(score == m, lane_n, _N))
        row = gather_row(idx)
        c = jnp.where(row_k == i, row, c)
        md = jnp.minimum(md, dist_row(row))
        return c, md

    c, _ = lax.fori_loop(1, _K, init_step, (c_init, dist_row(c0)))

    def assign(c):
        cx = _dot(c, x, ((1,), (1,)), lax.Precision.DEFAULT)
        c2 = jnp.sum(c * c, axis=1, keepdims=True)
        d = jnp.maximum(x2r - 2.0 * cx + c2, 0.0)
        md = jnp.min(d, axis=0, keepdims=True)
        lab = jnp.min(jnp.where(d == md, row_k, _K), axis=0, keepdims=True)
        return md, lab

    def lloyd_step(_, c):
        md, lab = assign(c)
        fm = jnp.max(md)
        fidx = jnp.min(jnp.where(md == fm, lane_n, _N))
        frow = gather_row(fidx)
        oh = jnp.where(lab == row_k, 1.0, 0.0)
        sums = _dot(oh, x, ((1,), (0,)))
        counts = jnp.sum(oh, axis=1, keepdims=True)
        safe = sums / jnp.maximum(counts, 1.0)
        return jnp.where(counts > 0, safe, frow)

    c = lax.fori_loop(0, _ITERS, lloyd_step, c)
    _, lab = assign(c)
    cent_ref[:] = c
    lab_ref[:] = lab


def _pallas_kmeans(flat, gmat, c0):
    return pl.pallas_call(
        _kmeans_kernel,
        out_shape=(jax.ShapeDtypeStruct((_K, _D), jnp.float32),
                   jax.ShapeDtypeStruct((1, _N), jnp.int32)),
        compiler_params=pltpu.CompilerParams(
            vmem_limit_bytes=100 * 1024 * 1024),
    )(flat, gmat, c0)


def kernel(data):
    flat = data.reshape(_N, _D)
    with jax.ensure_compile_time_eval():
        init_key = jax.random.PRNGKey(0)
        first = int(jax.random.choice(init_key, jnp.arange(_N), (1,),
                                      replace=False)[0])
        gmat = jnp.stack([
            jax.random.gumbel(jax.random.fold_in(init_key, i), (_N,),
                              jnp.float32)
            for i in range(1, _K)])
    c0 = flat[first][None, :]
    cent, lab = _pallas_kmeans(flat, gmat, c0)
    return cent, lab.reshape(_N)
